# baseline (speedup 1.0000x reference)
import functools

import jax
import jax.numpy as jnp
from jax import lax
from jax.experimental import pallas as pl
from jax.experimental.pallas import tpu as pltpu
from jax.experimental.pallas import tpu_sc as plsc

N = 10000
E = 320000
D_IN = 128
H = 16
D_OUT = 128
B = 100
T = 100

NC = 2
NS = 16
NP = 10240
EPT = 10240
EP = EPT * NC * NS
RPT = NP // NS
EPT_REAL = E // (NC * NS)
CH = 2048
NCHUNK = EPT // CH
SUBW = 64
SUB = CH // SUBW
SPG = SUBW // 16
GRP = CH // 16
TAIL_REAL = EPT_REAL - (NCHUNK - 1) * CH
TAIL_GRP = TAIL_REAL // 16


@functools.cache
def _mesh():
    return plsc.VectorSubcoreMesh(core_axis_name="c", subcore_axis_name="s",
                                  num_cores=NC, num_subcores=NS)


def _perm16(v):
    q = lax.shift_right_arithmetic(v * 10486, 20)
    r = v - q * T
    return r * B + q


def _sc_prep_body(ei_hbm, deg_hbm, si_hbm, di_hbm, acc, sbuf, dbuf, srcp,
                  dstb, onesbuf, zbuf, sems_a, sems_b, sem_w):
    cid = lax.axis_index("c")
    sid = lax.axis_index("s")

    def _zero(i, _):
        zbuf[pl.ds(i * 16, 16)] = jnp.zeros((16,), jnp.float32)
        return 0
    lax.fori_loop(0, RPT // 16, _zero, 0, unroll=True)

    def _ones(i, _):
        onesbuf[pl.ds(i * 16, 16)] = jnp.ones((16,), jnp.float32)
        return 0
    lax.fori_loop(0, SUBW // 16, _ones, 0, unroll=True)
    pltpu.sync_copy(zbuf, acc.at[pl.ds(sid * RPT, RPT)])
    plsc.subcore_barrier()

    tid = cid * NS + sid
    ebase = tid * EPT_REAL
    obase = tid * EPT
    orow = obase // SUBW
    sems = (sems_a, sems_b)

    def _stage(k, p):
        real = CH if k < NCHUNK - 1 else TAIL_REAL
        rgrp = GRP if k < NCHUNK - 1 else TAIL_GRP
        off = pl.multiple_of(ebase + k * CH, 16)
        pltpu.sync_copy(ei_hbm.at[0, pl.ds(off, real)],
                        sbuf.at[p, pl.ds(0, real)])
        pltpu.sync_copy(ei_hbm.at[1, pl.ds(off, real)],
                        dbuf.at[p, pl.ds(0, real)])

        def _fill(g, _):
            go = pl.multiple_of(g * 16, 16)
            srcp[p, pl.ds(go, 16)] = _perm16(sbuf[p, pl.ds(go, 16)])
            co = pl.multiple_of((g % SPG) * 16, 16)
            dstb[p, g // SPG, pl.ds(co, 16)] = _perm16(dbuf[p, pl.ds(go, 16)])
            return 0
        lax.fori_loop(0, rgrp, _fill, 0)
        if k == NCHUNK - 1:
            lane = lax.iota(jnp.int32, 16)
            for g in range(TAIL_GRP, GRP):
                srcp[p, pl.ds(g * 16, 16)] = jnp.zeros((16,), jnp.int32)
                dstb[p, g // SPG, pl.ds((g % SPG) * 16, 16)] = (
                    N + (g - TAIL_GRP) * 16 + lane)
        wd = [pltpu.async_copy(srcp.at[p],
                               si_hbm.at[pl.ds(obase + k * CH, CH)], sem_w),
              pltpu.async_copy(dstb.at[p],
                               di_hbm.at[pl.ds(orow + k * SUB, SUB)], sem_w)]
        return wd

    sc_descs = [None] * NCHUNK
    w_descs = [None] * NCHUNK
    w_descs[0] = _stage(0, 0)
    for k in range(NCHUNK):
        p = k % 2
        sc_descs[k] = [
            pltpu.async_copy(onesbuf, acc.at[dstb.at[p, j]], sems[p],
                             add=True)
            for j in range(SUB)]
        if k + 1 < NCHUNK:
            if k >= 1:
                for d in sc_descs[k - 1] + w_descs[k - 1]:
                    d.wait()
            w_descs[k + 1] = _stage(k + 1, 1 - p)
    for kk in (NCHUNK - 2, NCHUNK - 1):
        for d in sc_descs[kk] + w_descs[kk]:
            d.wait()

    plsc.subcore_barrier()
    pltpu.sync_copy(acc.at[pl.ds(sid * RPT, RPT)], zbuf)
    pltpu.sync_copy(zbuf, deg_hbm.at[cid, pl.ds(sid * RPT, RPT)])


@functools.cache
def _sc_prep_kernel():
    return pl.kernel(
        _sc_prep_body,
        out_type=[
            jax.ShapeDtypeStruct((NC, NP), jnp.float32),
            jax.ShapeDtypeStruct((EP,), jnp.int32),
            jax.ShapeDtypeStruct((EP // SUBW, SUBW), jnp.int32),
        ],
        mesh=_mesh(),
        compiler_params=pltpu.CompilerParams(use_tc_tiling_on_sc=False),
        scratch_types=[
            pltpu.VMEM_SHARED((NP,), jnp.float32),
            pltpu.VMEM((2, CH), jnp.int32),
            pltpu.VMEM((2, CH), jnp.int32),
            pltpu.VMEM((2, CH), jnp.int32),
            pltpu.VMEM((2, SUB, SUBW), jnp.int32),
            pltpu.VMEM((SUBW,), jnp.float32),
            pltpu.VMEM((RPT,), jnp.float32),
            pltpu.SemaphoreType.DMA,
            pltpu.SemaphoreType.DMA,
            pltpu.SemaphoreType.DMA,
        ],
    )


def _sc_prep(ei):
    return _sc_prep_kernel()(ei)


def _sc_agg_body(g_hbm, si_hbm, di_hbm, out_hbm, acc, srcb, dstb, rows,
                 zbuf, semg, sems_a, sems_b):
    cid = lax.axis_index("c")
    sid = lax.axis_index("s")

    def _zero(i, _):
        zbuf[i] = jnp.zeros((16,), jnp.float32)
        return 0
    lax.fori_loop(0, RPT, _zero, 0)
    pltpu.sync_copy(zbuf, acc.at[pl.ds(sid * RPT, RPT)])
    plsc.subcore_barrier()

    obase = (cid * NS + sid) * EPT
    orow = obase // SUBW
    sems = (sems_a, sems_b)

    def _stage(k, p):
        pltpu.sync_copy(si_hbm.at[pl.ds(obase + k * CH, CH)], srcb.at[p])
        pltpu.sync_copy(di_hbm.at[pl.ds(orow + k * SUB, SUB)], dstb.at[p])

    def _gather(p):
        return pltpu.async_copy(g_hbm.at[srcb.at[p]], rows.at[p], semg)

    g_descs = [None] * NCHUNK
    sc_descs = [None] * NCHUNK
    _stage(0, 0)
    g_descs[0] = _gather(0)
    for k in range(NCHUNK):
        p = k % 2
        g_descs[k].wait()
        if k + 1 < NCHUNK:
            if k >= 1:
                for d in sc_descs[k - 1]:
                    d.wait()
            _stage(k + 1, 1 - p)
            g_descs[k + 1] = _gather(1 - p)
        sc_descs[k] = [
            pltpu.async_copy(rows.at[p, pl.ds(j * SUBW, SUBW)],
                             acc.at[dstb.at[p, j]], sems[p], add=True)
            for j in range(SUB)]
    for kk in (NCHUNK - 2, NCHUNK - 1):
        for d in sc_descs[kk]:
            d.wait()

    plsc.subcore_barrier()
    pltpu.sync_copy(acc.at[pl.ds(sid * RPT, RPT)], zbuf)
    pltpu.sync_copy(zbuf, out_hbm.at[cid, pl.ds(sid * RPT, RPT)])


@functools.cache
def _sc_agg_kernel():
    return pl.kernel(
        _sc_agg_body,
        out_type=jax.ShapeDtypeStruct((NC, NP, H), jnp.float32),
        mesh=_mesh(),
        compiler_params=pltpu.CompilerParams(use_tc_tiling_on_sc=False),
        scratch_types=[
            pltpu.VMEM_SHARED((NP, H), jnp.float32),
            pltpu.VMEM((2, CH), jnp.int32),
            pltpu.VMEM((2, SUB, SUBW), jnp.int32),
            pltpu.VMEM((2, CH, H), jnp.float32),
            pltpu.VMEM((RPT, H), jnp.float32),
            pltpu.SemaphoreType.DMA,
            pltpu.SemaphoreType.DMA,
            pltpu.SemaphoreType.DMA,
        ],
    )


def _sc_agg(g, si, di):
    return _sc_agg_kernel()(g, si, di)


RB = 2000


def _tc_g1_body(x_ref, d0_ref, d1_ref, w1_ref, g1_ref, dinv_ref):
    dinv = lax.rsqrt(d0_ref[...] + d1_ref[...] + 1.0)
    g1_ref[...] = dinv * jnp.dot(x_ref[...], w1_ref[...],
                                 preferred_element_type=jnp.float32)
    dinv_ref[...] = dinv


def _tc_g1(x, degp3, W1):
    return pl.pallas_call(
        _tc_g1_body,
        grid=(N // RB,),
        in_specs=[
            pl.BlockSpec((RB, D_IN), lambda i: (i, 0)),
            pl.BlockSpec((None, RB, 1), lambda i: (0, i, 0)),
            pl.BlockSpec((None, RB, 1), lambda i: (1, i, 0)),
            pl.BlockSpec((D_IN, H), lambda i: (0, 0)),
        ],
        out_specs=[
            pl.BlockSpec((RB, H), lambda i: (i, 0)),
            pl.BlockSpec((RB, 1), lambda i: (i, 0)),
        ],
        out_shape=[
            jax.ShapeDtypeStruct((N, H), jnp.float32),
            jax.ShapeDtypeStruct((N, 1), jnp.float32),
        ],
    )(x, degp3, degp3, W1)


def _tc_g2_body(s0_ref, s1_ref, g1_ref, dinv_ref, w2_ref, b1_ref, out_ref):
    dinv = dinv_ref[...]
    h1 = jax.nn.relu(dinv * (s0_ref[...] + s1_ref[...] + g1_ref[...])
                     + b1_ref[...])
    out_ref[...] = dinv * jnp.dot(h1, w2_ref[...],
                                  preferred_element_type=jnp.float32)


def _tc_g2(s, g1, dinv, W2, b1):
    return pl.pallas_call(
        _tc_g2_body,
        grid=(N // RB,),
        in_specs=[
            pl.BlockSpec((None, RB, H), lambda i: (0, i, 0)),
            pl.BlockSpec((None, RB, H), lambda i: (1, i, 0)),
            pl.BlockSpec((RB, H), lambda i: (i, 0)),
            pl.BlockSpec((RB, 1), lambda i: (i, 0)),
            pl.BlockSpec((H, H), lambda i: (0, 0)),
            pl.BlockSpec((H,), lambda i: (0,)),
        ],
        out_specs=pl.BlockSpec((RB, H), lambda i: (i, 0)),
        out_shape=jax.ShapeDtypeStruct((N, H), jnp.float32),
    )(s, s, g1, dinv, W2, b1)


def _tc_gru_body(s0_ref, s1_ref, g2_ref, dinv_ref, b2_ref,
                 wih_ref, whh_ref, bih_ref, bhh_ref,
                 wfc_ref, bfc_ref, out_ref):
    pre = (s0_ref[...][:N] + s1_ref[...][:N] + g2_ref[...])
    xt = jax.nn.relu(dinv_ref[...] * pre + b2_ref[...])

    def mm(a, w):
        return jnp.dot(a, w, preferred_element_type=jnp.float32)

    wih = wih_ref[...]
    whh = whh_ref[...]
    bih = bih_ref[...]
    bhh = bhh_ref[...]
    gr = (mm(xt, wih[:, :H]) + bih[:H]).reshape(T, B, H)
    gz = (mm(xt, wih[:, H:2 * H]) + bih[H:2 * H]).reshape(T, B, H)
    gn = (mm(xt, wih[:, 2 * H:]) + bih[2 * H:]).reshape(T, B, H)
    whr, whz, whn = whh[:, :H], whh[:, H:2 * H], whh[:, 2 * H:]
    bhr, bhz, bhn = bhh[:H], bhh[H:2 * H], bhh[2 * H:]

    h = jnp.zeros((B, H), jnp.float32)
    for t in range(T):
        r = jax.nn.sigmoid(gr[t] + mm(h, whr) + bhr)
        z = jax.nn.sigmoid(gz[t] + mm(h, whz) + bhz)
        n = jnp.tanh(gn[t] + r * (mm(h, whn) + bhn))
        h = (1.0 - z) * n + z * h
    out_ref[...] = mm(h, wfc_ref[...]) + bfc_ref[...]


def _tc_gru(s, g2, dinv, b2, W_ih, W_hh, b_ih, b_hh, Wfc, bfc):
    return pl.pallas_call(
        _tc_gru_body,
        grid=(1,),
        in_specs=[
            pl.BlockSpec((None, NP, H), lambda i: (0, 0, 0)),
            pl.BlockSpec((None, NP, H), lambda i: (1, 0, 0)),
            pl.BlockSpec((N, H), lambda i: (0, 0)),
            pl.BlockSpec((N, 1), lambda i: (0, 0)),
            pl.BlockSpec((H,), lambda i: (0,)),
            pl.BlockSpec((H, 3 * H), lambda i: (0, 0)),
            pl.BlockSpec((H, 3 * H), lambda i: (0, 0)),
            pl.BlockSpec((3 * H,), lambda i: (0,)),
            pl.BlockSpec((3 * H,), lambda i: (0,)),
            pl.BlockSpec((H, D_OUT), lambda i: (0, 0)),
            pl.BlockSpec((D_OUT,), lambda i: (0,)),
        ],
        out_specs=pl.BlockSpec((B, D_OUT), lambda i: (0, 0)),
        out_shape=jax.ShapeDtypeStruct((B, D_OUT), jnp.float32),
    )(s, s, g2, dinv, b2, W_ih, W_hh, b_ih, b_hh, Wfc, bfc)


def kernel(x, edge_index, batch, W1, b1, W2, b2, W_ih, W_hh, b_ih, b_hh,
           Wfc, bfc):
    x = x.reshape(B, T, D_IN).swapaxes(0, 1).reshape(N, D_IN)

    degp, si, di = _sc_prep(edge_index)
    g1, dinv = _tc_g1(x, degp[:, :, None], W1)
    s = _sc_agg(g1, si, di)
    g2 = _tc_g2(s, g1, dinv, W2, b1)
    s2 = _sc_agg(g2, si, di)
    return _tc_gru(s2, g2, dinv, b2, W_ih, W_hh, b_ih, b_hh, Wfc, bfc)


# speedup vs baseline: 1.0273x; 1.0273x over previous
import functools

import jax
import jax.numpy as jnp
from jax import lax
from jax.experimental import pallas as pl
from jax.experimental.pallas import tpu as pltpu
from jax.experimental.pallas import tpu_sc as plsc

N = 10000
E = 320000
D_IN = 128
H = 16
D_OUT = 128
B = 100
T = 100

NC = 2
NS = 16
NP = 10240
EPT = 10240
EP = EPT * NC * NS
RPT = NP // NS
EPT_REAL = E // (NC * NS)
CH = 2048
NCHUNK = EPT // CH
SUBW = 128
SUB = CH // SUBW
SPG = SUBW // 16
GRP = CH // 16
TAIL_REAL = EPT_REAL - (NCHUNK - 1) * CH
TAIL_GRP = TAIL_REAL // 16


@functools.cache
def _mesh():
    return plsc.VectorSubcoreMesh(core_axis_name="c", subcore_axis_name="s",
                                  num_cores=NC, num_subcores=NS)


def _perm16(v):
    q = lax.shift_right_arithmetic(v * 10486, 20)
    r = v - q * T
    return r * B + q


def _sc_prep_body(ei_hbm, deg_hbm, si_hbm, di_hbm, acc, sbuf, dbuf, srcp,
                  dstb, onesbuf, zbuf, sems_a, sems_b, sem_w):
    cid = lax.axis_index("c")
    sid = lax.axis_index("s")

    def _zero(i, _):
        zbuf[pl.ds(i * 16, 16)] = jnp.zeros((16,), jnp.float32)
        return 0
    lax.fori_loop(0, RPT // 16, _zero, 0, unroll=True)

    def _ones(i, _):
        onesbuf[pl.ds(i * 16, 16)] = jnp.ones((16,), jnp.float32)
        return 0
    lax.fori_loop(0, SUBW // 16, _ones, 0, unroll=True)
    pltpu.sync_copy(zbuf, acc.at[pl.ds(sid * RPT, RPT)])
    plsc.subcore_barrier()

    tid = cid * NS + sid
    ebase = tid * EPT_REAL
    obase = tid * EPT
    orow = obase // SUBW
    sems = (sems_a, sems_b)

    def _stage(k, p):
        real = CH if k < NCHUNK - 1 else TAIL_REAL
        rgrp = GRP if k < NCHUNK - 1 else TAIL_GRP
        off = pl.multiple_of(ebase + k * CH, 16)
        pltpu.sync_copy(ei_hbm.at[0, pl.ds(off, real)],
                        sbuf.at[p, pl.ds(0, real)])
        pltpu.sync_copy(ei_hbm.at[1, pl.ds(off, real)],
                        dbuf.at[p, pl.ds(0, real)])

        def _fill(g, _):
            go = pl.multiple_of(g * 16, 16)
            srcp[p, pl.ds(go, 16)] = _perm16(sbuf[p, pl.ds(go, 16)])
            co = pl.multiple_of((g % SPG) * 16, 16)
            dstb[p, g // SPG, pl.ds(co, 16)] = _perm16(dbuf[p, pl.ds(go, 16)])
            return 0
        lax.fori_loop(0, rgrp, _fill, 0)
        if k == NCHUNK - 1:
            lane = lax.iota(jnp.int32, 16)
            for g in range(TAIL_GRP, GRP):
                srcp[p, pl.ds(g * 16, 16)] = jnp.zeros((16,), jnp.int32)
                dstb[p, g // SPG, pl.ds((g % SPG) * 16, 16)] = (
                    N + (g - TAIL_GRP) * 16 + lane)
        wd = [pltpu.async_copy(srcp.at[p],
                               si_hbm.at[pl.ds(obase + k * CH, CH)], sem_w),
              pltpu.async_copy(dstb.at[p],
                               di_hbm.at[pl.ds(orow + k * SUB, SUB)], sem_w)]
        return wd

    sc_descs = [None] * NCHUNK
    w_descs = [None] * NCHUNK
    w_descs[0] = _stage(0, 0)
    for k in range(NCHUNK):
        p = k % 2
        sc_descs[k] = [
            pltpu.async_copy(onesbuf, acc.at[dstb.at[p, j]], sems[p],
                             add=True)
            for j in range(SUB)]
        if k + 1 < NCHUNK:
            if k >= 1:
                for d in sc_descs[k - 1] + w_descs[k - 1]:
                    d.wait()
            w_descs[k + 1] = _stage(k + 1, 1 - p)
    for kk in (NCHUNK - 2, NCHUNK - 1):
        for d in sc_descs[kk] + w_descs[kk]:
            d.wait()

    plsc.subcore_barrier()
    pltpu.sync_copy(acc.at[pl.ds(sid * RPT, RPT)], zbuf)
    pltpu.sync_copy(zbuf, deg_hbm.at[cid, pl.ds(sid * RPT, RPT)])


@functools.cache
def _sc_prep_kernel():
    return pl.kernel(
        _sc_prep_body,
        out_type=[
            jax.ShapeDtypeStruct((NC, NP), jnp.float32),
            jax.ShapeDtypeStruct((EP,), jnp.int32),
            jax.ShapeDtypeStruct((EP // SUBW, SUBW), jnp.int32),
        ],
        mesh=_mesh(),
        compiler_params=pltpu.CompilerParams(use_tc_tiling_on_sc=False),
        scratch_types=[
            pltpu.VMEM_SHARED((NP,), jnp.float32),
            pltpu.VMEM((2, CH), jnp.int32),
            pltpu.VMEM((2, CH), jnp.int32),
            pltpu.VMEM((2, CH), jnp.int32),
            pltpu.VMEM((2, SUB, SUBW), jnp.int32),
            pltpu.VMEM((SUBW,), jnp.float32),
            pltpu.VMEM((RPT,), jnp.float32),
            pltpu.SemaphoreType.DMA,
            pltpu.SemaphoreType.DMA,
            pltpu.SemaphoreType.DMA,
        ],
    )


def _sc_prep(ei):
    return _sc_prep_kernel()(ei)


def _sc_agg_body(g_hbm, si_hbm, di_hbm, out_hbm, acc, srcb, dstb, rows,
                 zbuf, semg, sems_a, sems_b):
    cid = lax.axis_index("c")
    sid = lax.axis_index("s")

    def _zero(i, _):
        zbuf[i] = jnp.zeros((16,), jnp.float32)
        return 0
    lax.fori_loop(0, RPT, _zero, 0)
    pltpu.sync_copy(zbuf, acc.at[pl.ds(sid * RPT, RPT)])
    plsc.subcore_barrier()

    obase = (cid * NS + sid) * EPT
    orow = obase // SUBW
    sems = (sems_a, sems_b)

    def _stage(k, p):
        pltpu.sync_copy(si_hbm.at[pl.ds(obase + k * CH, CH)], srcb.at[p])
        pltpu.sync_copy(di_hbm.at[pl.ds(orow + k * SUB, SUB)], dstb.at[p])

    def _gather(p):
        return pltpu.async_copy(g_hbm.at[srcb.at[p]], rows.at[p], semg)

    g_descs = [None] * NCHUNK
    sc_descs = [None] * NCHUNK
    _stage(0, 0)
    g_descs[0] = _gather(0)
    for k in range(NCHUNK):
        p = k % 2
        g_descs[k].wait()
        if k + 1 < NCHUNK:
            if k >= 1:
                for d in sc_descs[k - 1]:
                    d.wait()
            _stage(k + 1, 1 - p)
            g_descs[k + 1] = _gather(1 - p)
        sc_descs[k] = [
            pltpu.async_copy(rows.at[p, pl.ds(j * SUBW, SUBW)],
                             acc.at[dstb.at[p, j]], sems[p], add=True)
            for j in range(SUB)]
    for kk in (NCHUNK - 2, NCHUNK - 1):
        for d in sc_descs[kk]:
            d.wait()

    plsc.subcore_barrier()
    pltpu.sync_copy(acc.at[pl.ds(sid * RPT, RPT)], zbuf)
    pltpu.sync_copy(zbuf, out_hbm.at[cid, pl.ds(sid * RPT, RPT)])


@functools.cache
def _sc_agg_kernel():
    return pl.kernel(
        _sc_agg_body,
        out_type=jax.ShapeDtypeStruct((NC, NP, H), jnp.float32),
        mesh=_mesh(),
        compiler_params=pltpu.CompilerParams(use_tc_tiling_on_sc=False),
        scratch_types=[
            pltpu.VMEM_SHARED((NP, H), jnp.float32),
            pltpu.VMEM((2, CH), jnp.int32),
            pltpu.VMEM((2, SUB, SUBW), jnp.int32),
            pltpu.VMEM((2, CH, H), jnp.float32),
            pltpu.VMEM((RPT, H), jnp.float32),
            pltpu.SemaphoreType.DMA,
            pltpu.SemaphoreType.DMA,
            pltpu.SemaphoreType.DMA,
        ],
    )


def _sc_agg(g, si, di):
    return _sc_agg_kernel()(g, si, di)


RB = 2000


def _tc_g1_body(x_ref, d0_ref, d1_ref, w1_ref, g1_ref, dinv_ref):
    dinv = lax.rsqrt(d0_ref[...] + d1_ref[...] + 1.0)
    g1_ref[...] = dinv * jnp.dot(x_ref[...], w1_ref[...],
                                 preferred_element_type=jnp.float32)
    dinv_ref[...] = dinv


def _tc_g1(x, d0, d1, W1):
    return pl.pallas_call(
        _tc_g1_body,
        grid=(N // RB,),
        in_specs=[
            pl.BlockSpec((RB, D_IN), lambda i: (i, 0)),
            pl.BlockSpec((RB, 1), lambda i: (i, 0)),
            pl.BlockSpec((RB, 1), lambda i: (i, 0)),
            pl.BlockSpec((D_IN, H), lambda i: (0, 0)),
        ],
        out_specs=[
            pl.BlockSpec((RB, H), lambda i: (i, 0)),
            pl.BlockSpec((RB, 1), lambda i: (i, 0)),
        ],
        out_shape=[
            jax.ShapeDtypeStruct((N, H), jnp.float32),
            jax.ShapeDtypeStruct((N, 1), jnp.float32),
        ],
    )(x, d0, d1, W1)


def _tc_g2_body(s0_ref, s1_ref, g1_ref, dinv_ref, w2_ref, b1_ref, out_ref):
    dinv = dinv_ref[...]
    h1 = jax.nn.relu(dinv * (s0_ref[...] + s1_ref[...] + g1_ref[...])
                     + b1_ref[...])
    out_ref[...] = dinv * jnp.dot(h1, w2_ref[...],
                                  preferred_element_type=jnp.float32)


def _tc_g2(s, g1, dinv, W2, b1):
    return pl.pallas_call(
        _tc_g2_body,
        grid=(N // RB,),
        in_specs=[
            pl.BlockSpec((None, RB, H), lambda i: (0, i, 0)),
            pl.BlockSpec((None, RB, H), lambda i: (1, i, 0)),
            pl.BlockSpec((RB, H), lambda i: (i, 0)),
            pl.BlockSpec((RB, 1), lambda i: (i, 0)),
            pl.BlockSpec((H, H), lambda i: (0, 0)),
            pl.BlockSpec((H,), lambda i: (0,)),
        ],
        out_specs=pl.BlockSpec((RB, H), lambda i: (i, 0)),
        out_shape=jax.ShapeDtypeStruct((N, H), jnp.float32),
    )(s, s, g1, dinv, W2, b1)


def _tc_gru_body(s0_ref, s1_ref, g2_ref, dinv_ref, b2_ref,
                 wih_ref, whh_ref, bih_ref, bhh_ref,
                 wfc_ref, bfc_ref, out_ref):
    pre = (s0_ref[...][:N] + s1_ref[...][:N] + g2_ref[...])
    xt = jax.nn.relu(dinv_ref[...] * pre + b2_ref[...])

    def mm(a, w):
        return jnp.dot(a, w, preferred_element_type=jnp.float32)

    wih = wih_ref[...]
    whh = whh_ref[...]
    bih = bih_ref[...]
    bhh = bhh_ref[...]
    gr = (mm(xt, wih[:, :H]) + bih[:H]).reshape(T, B, H)
    gz = (mm(xt, wih[:, H:2 * H]) + bih[H:2 * H]).reshape(T, B, H)
    gn = (mm(xt, wih[:, 2 * H:]) + bih[2 * H:]).reshape(T, B, H)
    whr, whz, whn = whh[:, :H], whh[:, H:2 * H], whh[:, 2 * H:]
    bhr, bhz, bhn = bhh[:H], bhh[H:2 * H], bhh[2 * H:]

    h = jnp.zeros((B, H), jnp.float32)
    for t in range(T):
        r = jax.nn.sigmoid(gr[t] + mm(h, whr) + bhr)
        z = jax.nn.sigmoid(gz[t] + mm(h, whz) + bhz)
        n = jnp.tanh(gn[t] + r * (mm(h, whn) + bhn))
        h = (1.0 - z) * n + z * h
    out_ref[...] = mm(h, wfc_ref[...]) + bfc_ref[...]


def _tc_gru(s, g2, dinv, b2, W_ih, W_hh, b_ih, b_hh, Wfc, bfc):
    return pl.pallas_call(
        _tc_gru_body,
        grid=(1,),
        in_specs=[
            pl.BlockSpec((None, NP, H), lambda i: (0, 0, 0)),
            pl.BlockSpec((None, NP, H), lambda i: (1, 0, 0)),
            pl.BlockSpec((N, H), lambda i: (0, 0)),
            pl.BlockSpec((N, 1), lambda i: (0, 0)),
            pl.BlockSpec((H,), lambda i: (0,)),
            pl.BlockSpec((H, 3 * H), lambda i: (0, 0)),
            pl.BlockSpec((H, 3 * H), lambda i: (0, 0)),
            pl.BlockSpec((3 * H,), lambda i: (0,)),
            pl.BlockSpec((3 * H,), lambda i: (0,)),
            pl.BlockSpec((H, D_OUT), lambda i: (0, 0)),
            pl.BlockSpec((D_OUT,), lambda i: (0,)),
        ],
        out_specs=pl.BlockSpec((B, D_OUT), lambda i: (0, 0)),
        out_shape=jax.ShapeDtypeStruct((B, D_OUT), jnp.float32),
    )(s, s, g2, dinv, b2, W_ih, W_hh, b_ih, b_hh, Wfc, bfc)


def kernel(x, edge_index, batch, W1, b1, W2, b2, W_ih, W_hh, b_ih, b_hh,
           Wfc, bfc):
    x = x.reshape(B, T, D_IN).swapaxes(0, 1).reshape(N, D_IN)

    degp, si, di = _sc_prep(edge_index)
    g1, dinv = _tc_g1(x, degp[0].reshape(NP, 1), degp[1].reshape(NP, 1), W1)
    s = _sc_agg(g1, si, di)
    g2 = _tc_g2(s, g1, dinv, W2, b1)
    s2 = _sc_agg(g2, si, di)
    return _tc_gru(s2, g2, dinv, b2, W_ih, W_hh, b_ih, b_hh, Wfc, bfc)
